# baseline (speedup 1.0000x reference)
import functools

import jax
import jax.numpy as jnp
from jax import lax
from jax.experimental import pallas as pl
from jax.experimental.pallas import tpu as pltpu
from jax.experimental.pallas import tpu_sc as plsc

S = 128
N = 558
B2 = 4096
DIM = 128
AROWS = 560
ACOLS = 640
NPLANES = ACOLS // 128
AFLAT = AROWS * ACOLS
AVROWS = AFLAT // 128
NTILES = 16
SPS = S // 2
CHUNKS = B2 // NTILES // 128
SLICE = AFLAT // NTILES


def _prep_body(src_ref, dst_ref, bd_ref, w_ref, fidx_ref):
    src = src_ref[...]
    dst = dst_ref[...]
    bd = bd_ref[...]
    neq = (src != dst).astype(jnp.int32)
    b = jnp.sum(neq, axis=1, keepdims=True)
    pos = lax.broadcasted_iota(jnp.int32, (S, B2), 1)
    mask = pos < b
    masked = jnp.where(mask, bd, -jnp.inf)
    m = jnp.max(masked)
    w_ref[...] = jnp.where(mask, bd / m, jnp.zeros_like(bd))
    fidx_ref[...] = (
        (src // 128) * (AROWS * 128) + dst * 128 + (src % 128)
    )


def _prep(src, dst, bd):
    return pl.pallas_call(
        _prep_body,
        out_shape=(
            jax.ShapeDtypeStruct((S, B2), jnp.float32),
            jax.ShapeDtypeStruct((S, B2), jnp.int32),
        ),
    )(src, dst, bd)


def _sc_body(fidx_hbm, w_hbm, a_hbm, idx_v, w_v, negw_v, zbuf, a_sh):
    c = lax.axis_index("c")
    sid = lax.axis_index("s")

    def _zero(i, carry):
        zbuf[pl.ds(i * 16, 16)] = jnp.zeros((16,), jnp.float32)
        return carry

    lax.fori_loop(0, SLICE // 16, _zero, 0)
    a_flat = a_sh
    pltpu.sync_copy(zbuf, a_flat.at[pl.ds(sid * SLICE, SLICE)])
    plsc.subcore_barrier()

    def _sample(si, carry):
        s = c * SPS + si
        pltpu.sync_copy(fidx_hbm.at[s, pl.ds(sid * CHUNKS, CHUNKS)], idx_v)
        pltpu.sync_copy(w_hbm.at[s, pl.ds(sid * CHUNKS, CHUNKS)], w_v)
        for j in range(CHUNKS):
            for k in range(128 // 16):
                negw_v[j, pl.ds(k * 16, 16)] = -w_v[j, pl.ds(k * 16, 16)]
        for j in range(CHUNKS):
            pltpu.sync_copy(w_v.at[j], a_flat.at[idx_v.at[j]], add=True)
        plsc.subcore_barrier()
        pltpu.sync_copy(
            a_flat.at[pl.ds(sid * SLICE, SLICE)],
            a_hbm.at[s, pl.ds(sid * SLICE, SLICE)],
        )
        plsc.subcore_barrier()
        for j in range(CHUNKS):
            pltpu.sync_copy(negw_v.at[j], a_flat.at[idx_v.at[j]], add=True)
        return carry

    lax.fori_loop(0, SPS, _sample, 0)


def _sc_scatter(fidx, w):
    mesh = plsc.VectorSubcoreMesh(core_axis_name="c", subcore_axis_name="s")
    kfn = functools.partial(
        pl.kernel,
        mesh=mesh,
        out_type=jax.ShapeDtypeStruct((S, AFLAT), jnp.float32),
        scratch_types=[
            pltpu.VMEM((CHUNKS, 128), jnp.int32),
            pltpu.VMEM((CHUNKS, 128), jnp.float32),
            pltpu.VMEM((CHUNKS, 128), jnp.float32),
            pltpu.VMEM((SLICE,), jnp.float32),
            pltpu.VMEM_SHARED((AFLAT,), jnp.float32),
        ],
    )(_sc_body)
    return kfn(fidx, w)


def _gcn_body(x_ref, a_ref, w_ref, b_ref, o_ref):
    x = x_ref[0]
    a = a_ref[0]
    h = jnp.dot(x, w_ref[...], preferred_element_type=jnp.float32)
    hp = jnp.concatenate([h, jnp.zeros((AROWS - N, DIM), jnp.float32)], axis=0)
    planes = [a[j * AROWS:(j + 1) * AROWS] for j in range(NPLANES)]
    deg = 1.0 + jnp.sum(sum(planes[1:], planes[0]), axis=1)
    dis = lax.rsqrt(deg)
    t = hp * dis[:, None]
    z = jnp.zeros((AROWS, DIM), jnp.float32)
    for j in range(NPLANES):
        tj = lax.slice(
            jnp.concatenate([t, jnp.zeros((ACOLS - AROWS, DIM), jnp.float32)], axis=0),
            (j * 128, 0), (j * 128 + 128, DIM),
        )
        z = z + jnp.dot(planes[j], tj, preferred_element_type=jnp.float32)
    out = dis[:, None] * (z + t) + b_ref[...]
    o_ref[0] = out[:N]


def _gcn_tc(x_data, a, W, bias2d):
    return pl.pallas_call(
        _gcn_body,
        grid=(S,),
        in_specs=[
            pl.BlockSpec((1, N, DIM), lambda i: (i, 0, 0)),
            pl.BlockSpec((1, AVROWS, 128), lambda i: (i, 0, 0)),
            pl.BlockSpec((DIM, DIM), lambda i: (0, 0)),
            pl.BlockSpec((1, DIM), lambda i: (0, 0)),
        ],
        out_specs=pl.BlockSpec((1, N, DIM), lambda i: (i, 0, 0)),
        out_shape=jax.ShapeDtypeStruct((S, N, DIM), jnp.float32),
    )(x_data, a, W, bias2d)


@jax.jit
def kernel(x_data, edge_index, bond_dist, W, bias):
    src = edge_index[:, 0, :].astype(jnp.int32)
    dst = edge_index[:, 1, :].astype(jnp.int32)
    w, fidx = _prep(src, dst, bond_dist)
    a_flat = _sc_scatter(
        fidx.reshape(S, B2 // 128, 128), w.reshape(S, B2 // 128, 128)
    )
    a = a_flat.reshape(S, AVROWS, 128)
    out = _gcn_tc(x_data, a, W, bias.reshape(1, DIM))
    return out.reshape(S * N, DIM)


# speedup vs baseline: 1.1801x; 1.1801x over previous
import functools

import jax
import jax.numpy as jnp
from jax import lax
from jax.experimental import pallas as pl
from jax.experimental.pallas import tpu as pltpu
from jax.experimental.pallas import tpu_sc as plsc

S = 128
N = 558
B2 = 4096
DIM = 128
AROWS = 560
ACOLS = 640
NPLANES = ACOLS // 128
AFLAT = AROWS * ACOLS
AVROWS = AFLAT // 128
NTILES = 16
SPS = S // 2
CHUNKS = B2 // NTILES // 128
SLICE = AFLAT // NTILES


def _prep_body(src_ref, dst_ref, bd_ref, w_ref, fidx_ref):
    src = src_ref[...]
    dst = dst_ref[...]
    bd = bd_ref[...]
    neq = (src != dst).astype(jnp.int32)
    b = jnp.sum(neq, axis=1, keepdims=True)
    pos = lax.broadcasted_iota(jnp.int32, (S, B2), 1)
    mask = pos < b
    masked = jnp.where(mask, bd, -jnp.inf)
    m = jnp.max(masked)
    w_ref[...] = jnp.where(mask, bd / m, jnp.zeros_like(bd))
    fidx_ref[...] = (
        (src // 128) * (AROWS * 128) + dst * 128 + (src % 128)
    )


def _prep(src, dst, bd):
    return pl.pallas_call(
        _prep_body,
        out_shape=(
            jax.ShapeDtypeStruct((S, B2), jnp.float32),
            jax.ShapeDtypeStruct((S, B2), jnp.int32),
        ),
    )(src, dst, bd)


def _sc_body(fidx_hbm, w_hbm, a_hbm, idx_v, w_v, negw_v, zbuf, a_sh):
    c = lax.axis_index("c")
    sid = lax.axis_index("s")

    def _zero(i, carry):
        zbuf[pl.ds(i * 16, 16)] = jnp.zeros((16,), jnp.float32)
        return carry

    lax.fori_loop(0, SLICE // 16, _zero, 0)
    a_flat = a_sh
    pltpu.sync_copy(zbuf, a_flat.at[pl.ds(sid * SLICE, SLICE)])
    plsc.subcore_barrier()

    def _sample(si, carry):
        s = c * SPS + si
        pltpu.sync_copy(fidx_hbm.at[s, pl.ds(sid * CHUNKS, CHUNKS)], idx_v)
        pltpu.sync_copy(w_hbm.at[s, pl.ds(sid * CHUNKS, CHUNKS)], w_v)
        for j in range(CHUNKS):
            for k in range(128 // 16):
                negw_v[j, pl.ds(k * 16, 16)] = -w_v[j, pl.ds(k * 16, 16)]
        for j in range(CHUNKS):
            pltpu.sync_copy(w_v.at[j], a_flat.at[idx_v.at[j]], add=True)
        plsc.subcore_barrier()
        pltpu.sync_copy(
            a_flat.at[pl.ds(sid * SLICE, SLICE)],
            a_hbm.at[s, pl.ds(sid * SLICE, SLICE)],
        )
        plsc.subcore_barrier()
        for j in range(CHUNKS):
            pltpu.sync_copy(negw_v.at[j], a_flat.at[idx_v.at[j]], add=True)
        return carry

    lax.fori_loop(0, SPS, _sample, 0)


def _sc_scatter(fidx, w):
    mesh = plsc.VectorSubcoreMesh(core_axis_name="c", subcore_axis_name="s")
    kfn = functools.partial(
        pl.kernel,
        mesh=mesh,
        out_type=jax.ShapeDtypeStruct((S, AFLAT), jnp.float32),
        scratch_types=[
            pltpu.VMEM((CHUNKS, 128), jnp.int32),
            pltpu.VMEM((CHUNKS, 128), jnp.float32),
            pltpu.VMEM((CHUNKS, 128), jnp.float32),
            pltpu.VMEM((SLICE,), jnp.float32),
            pltpu.VMEM_SHARED((AFLAT,), jnp.float32),
        ],
    )(_sc_body)
    return kfn(fidx, w)


SPB = 4


def _gcn_body(x_ref, a_ref, w_ref, b_ref, o_ref):
    for u in range(SPB):
        x = x_ref[u]
        a = a_ref[u]
        h = jnp.dot(x, w_ref[...], preferred_element_type=jnp.float32)
        hp = jnp.concatenate(
            [h, jnp.zeros((AROWS - N, DIM), jnp.float32)], axis=0
        )
        planes = [a[j * AROWS:(j + 1) * AROWS] for j in range(NPLANES)]
        deg = 1.0 + jnp.sum(sum(planes[1:], planes[0]), axis=1)
        dis = lax.rsqrt(deg)
        t = hp * dis[:, None]
        tp = jnp.concatenate(
            [t, jnp.zeros((ACOLS - AROWS, DIM), jnp.float32)], axis=0
        )
        z = jnp.zeros((AROWS, DIM), jnp.float32)
        for j in range(NPLANES):
            tj = lax.slice(tp, (j * 128, 0), (j * 128 + 128, DIM))
            z = z + jnp.dot(planes[j], tj, preferred_element_type=jnp.float32)
        out = dis[:, None] * (z + t) + b_ref[...]
        o_ref[pl.ds(u * N, N), :] = out[:N]


def _gcn_tc(x_data, a, W, bias2d):
    return pl.pallas_call(
        _gcn_body,
        grid=(S // SPB,),
        in_specs=[
            pl.BlockSpec((SPB, N, DIM), lambda i: (i, 0, 0)),
            pl.BlockSpec((SPB, AVROWS, 128), lambda i: (i, 0, 0)),
            pl.BlockSpec((DIM, DIM), lambda i: (0, 0)),
            pl.BlockSpec((1, DIM), lambda i: (0, 0)),
        ],
        out_specs=pl.BlockSpec((SPB * N, DIM), lambda i: (i, 0)),
        out_shape=jax.ShapeDtypeStruct((S * N, DIM), jnp.float32),
    )(x_data, a, W, bias2d)


@jax.jit
def kernel(x_data, edge_index, bond_dist, W, bias):
    src = edge_index[:, 0, :].astype(jnp.int32)
    dst = edge_index[:, 1, :].astype(jnp.int32)
    w, fidx = _prep(src, dst, bond_dist)
    a_flat = _sc_scatter(
        fidx.reshape(S, B2 // 128, 128), w.reshape(S, B2 // 128, 128)
    )
    a = a_flat.reshape(S, AVROWS, 128)
    return _gcn_tc(x_data, a, W, bias.reshape(1, DIM))


# speedup vs baseline: 1.5336x; 1.2996x over previous
import functools

import jax
import jax.numpy as jnp
from jax import lax
from jax.experimental import pallas as pl
from jax.experimental.pallas import tpu as pltpu
from jax.experimental.pallas import tpu_sc as plsc

S = 128
N = 558
B2 = 4096
DIM = 128
AROWS = 560
ACOLS = 640
NPLANES = ACOLS // 128
AFLAT = AROWS * ACOLS
AVROWS = AFLAT // 128
NTILES = 16
SPS = S // 2
CHUNKS = B2 // NTILES // 128
SLICE = AFLAT // NTILES


def _prep_body(src_ref, dst_ref, bd_ref, w_ref, fidx_ref):
    src = src_ref[...]
    dst = dst_ref[...]
    bd = bd_ref[...]
    neq = (src != dst).astype(jnp.int32)
    b = jnp.sum(neq, axis=1, keepdims=True)
    pos = lax.broadcasted_iota(jnp.int32, (S, B2), 1)
    mask = pos < b
    masked = jnp.where(mask, bd, -jnp.inf)
    m = jnp.max(masked)
    w_ref[...] = jnp.where(mask, bd / m, jnp.zeros_like(bd))
    fidx_ref[...] = (
        (src // 128) * (AROWS * 128) + dst * 128 + (src % 128)
    )


def _prep(src, dst, bd):
    return pl.pallas_call(
        _prep_body,
        out_shape=(
            jax.ShapeDtypeStruct((S, B2), jnp.float32),
            jax.ShapeDtypeStruct((S, B2), jnp.int32),
        ),
    )(src, dst, bd)


def _sc_body(fidx_hbm, w_hbm, a_hbm, idx_v, w_v, zbuf,
             a_sh0, a_sh1, sin0, sin1, sz0, sz1, swo0, swo1, ssc):
    c = lax.axis_index("c")
    sid = lax.axis_index("s")
    abufs = (a_sh0, a_sh1)
    sins = (sin0, sin1)
    szs = (sz0, sz1)
    swos = (swo0, swo1)
    myslice = pl.ds(sid * SLICE, SLICE)

    def _in_copies(s, slot, sem):
        return (
            pltpu.make_async_copy(
                fidx_hbm.at[s, pl.ds(sid * CHUNKS, CHUNKS)], idx_v.at[slot], sem
            ),
            pltpu.make_async_copy(
                w_hbm.at[s, pl.ds(sid * CHUNKS, CHUNKS)], w_v.at[slot], sem
            ),
        )

    def _zinit(i, carry):
        zbuf[pl.ds(i * 16, 16)] = jnp.zeros((16,), jnp.float32)
        return carry

    lax.fori_loop(0, SLICE // 16, _zinit, 0)
    pltpu.async_copy(zbuf, a_sh0.at[myslice], sz0)
    pltpu.async_copy(zbuf, a_sh1.at[myslice], sz1)
    for d in _in_copies(c * SPS, 0, sin0):
        d.start()

    def _step(si, b, not_first, not_last):
        ab = abufs[b]
        o = 1 - b
        for d in _in_copies(si, b, sins[b]):
            d.wait()

        @pl.when(not_last)
        def _():
            for d in _in_copies(si + 1, o, sins[o]):
                d.start()

        pltpu.make_async_copy(zbuf, ab.at[myslice], szs[b]).wait()
        plsc.subcore_barrier()
        for j in range(CHUNKS):
            pltpu.async_copy(w_v.at[b, j], ab.at[idx_v.at[b, j]], ssc, add=True)
        for j in range(CHUNKS):
            pltpu.make_async_copy(w_v.at[b, j], ab.at[idx_v.at[b, j]], ssc).wait()
        plsc.subcore_barrier()
        pltpu.async_copy(ab.at[myslice], a_hbm.at[si, myslice], swos[b])

        @pl.when(not_first)
        def _():
            pltpu.make_async_copy(
                abufs[o].at[myslice], a_hbm.at[si - 1, myslice], swos[o]
            ).wait()

        @pl.when(jnp.logical_and(not_first, not_last))
        def _():
            pltpu.async_copy(zbuf, abufs[o].at[myslice], szs[o])

    def _pair(k, carry):
        si0 = c * SPS + 2 * k
        _step(si0, 0, k > 0, jnp.bool_(True))
        _step(si0 + 1, 1, jnp.bool_(True), k < SPS // 2 - 1)
        return carry

    lax.fori_loop(0, SPS // 2, _pair, 0)
    bl = (SPS - 1) % 2
    pltpu.make_async_copy(
        abufs[bl].at[myslice], a_hbm.at[c * SPS + SPS - 1, myslice], swos[bl]
    ).wait()


def _sc_scatter(fidx, w):
    mesh = plsc.VectorSubcoreMesh(core_axis_name="c", subcore_axis_name="s")
    kfn = functools.partial(
        pl.kernel,
        mesh=mesh,
        out_type=jax.ShapeDtypeStruct((S, AFLAT), jnp.float32),
        scratch_types=[
            pltpu.VMEM((2, CHUNKS, 128), jnp.int32),
            pltpu.VMEM((2, CHUNKS, 128), jnp.float32),
            pltpu.VMEM((SLICE,), jnp.float32),
            pltpu.VMEM_SHARED((AFLAT,), jnp.float32),
            pltpu.VMEM_SHARED((AFLAT,), jnp.float32),
            pltpu.SemaphoreType.DMA,
            pltpu.SemaphoreType.DMA,
            pltpu.SemaphoreType.DMA,
            pltpu.SemaphoreType.DMA,
            pltpu.SemaphoreType.DMA,
            pltpu.SemaphoreType.DMA,
            pltpu.SemaphoreType.DMA,
        ],
    )(_sc_body)
    return kfn(fidx, w)


SPB = 4


def _gcn_body(x_ref, a_ref, w_ref, b_ref, o_ref):
    for u in range(SPB):
        x = x_ref[u]
        a = a_ref[u]
        h = jnp.dot(x, w_ref[...], preferred_element_type=jnp.float32)
        hp = jnp.concatenate(
            [h, jnp.zeros((AROWS - N, DIM), jnp.float32)], axis=0
        )
        planes = [a[j * AROWS:(j + 1) * AROWS] for j in range(NPLANES)]
        deg = 1.0 + jnp.sum(sum(planes[1:], planes[0]), axis=1)
        dis = lax.rsqrt(deg)
        t = hp * dis[:, None]
        tp = jnp.concatenate(
            [t, jnp.zeros((ACOLS - AROWS, DIM), jnp.float32)], axis=0
        )
        z = jnp.zeros((AROWS, DIM), jnp.float32)
        for j in range(NPLANES):
            tj = lax.slice(tp, (j * 128, 0), (j * 128 + 128, DIM))
            z = z + jnp.dot(planes[j], tj, preferred_element_type=jnp.float32)
        out = dis[:, None] * (z + t) + b_ref[...]
        o_ref[pl.ds(u * N, N), :] = out[:N]


def _gcn_tc(x_data, a, W, bias2d):
    return pl.pallas_call(
        _gcn_body,
        grid=(S // SPB,),
        in_specs=[
            pl.BlockSpec((SPB, N, DIM), lambda i: (i, 0, 0)),
            pl.BlockSpec((SPB, AVROWS, 128), lambda i: (i, 0, 0)),
            pl.BlockSpec((DIM, DIM), lambda i: (0, 0)),
            pl.BlockSpec((1, DIM), lambda i: (0, 0)),
        ],
        out_specs=pl.BlockSpec((SPB * N, DIM), lambda i: (i, 0)),
        out_shape=jax.ShapeDtypeStruct((S * N, DIM), jnp.float32),
    )(x_data, a, W, bias2d)


@jax.jit
def kernel(x_data, edge_index, bond_dist, W, bias):
    src = edge_index[:, 0, :].astype(jnp.int32)
    dst = edge_index[:, 1, :].astype(jnp.int32)
    w, fidx = _prep(src, dst, bond_dist)
    a_flat = _sc_scatter(
        fidx.reshape(S, B2 // 128, 128), w.reshape(S, B2 // 128, 128)
    )
    a = a_flat.reshape(S, AVROWS, 128)
    return _gcn_tc(x_data, a, W, bias.reshape(1, DIM))


# speedup vs baseline: 2.4292x; 1.5840x over previous
import functools

import jax
import jax.numpy as jnp
from jax import lax
from jax.experimental import pallas as pl
from jax.experimental.pallas import tpu as pltpu
from jax.experimental.pallas import tpu_sc as plsc

S = 128
N = 558
B2 = 4096
DIM = 128
AROWS = 560
ACOLS = 640
NPLANES = ACOLS // 128
AFLAT = AROWS * ACOLS
AVROWS = AFLAT // 128
NTILES = 16
SPS = S // 2
CHUNKS = B2 // NTILES // 128
SLICE = AFLAT // NTILES


def _prep_body(src_ref, dst_ref, bd_ref, w_ref, fidx_ref):
    src = src_ref[...]
    dst = dst_ref[...]
    bd = bd_ref[...]
    neq = (src != dst).astype(jnp.int32)
    b = jnp.sum(neq, axis=1, keepdims=True)
    pos = lax.broadcasted_iota(jnp.int32, (S, B2), 1)
    mask = pos < b
    masked = jnp.where(mask, bd, -jnp.inf)
    m = jnp.max(masked)
    w_ref[...] = jnp.where(mask, bd / m, jnp.zeros_like(bd))
    fidx_ref[...] = (
        (src // 128) * (AROWS * 128) + dst * 128 + (src % 128)
    )


def _prep(src, dst, bd):
    return pl.pallas_call(
        _prep_body,
        out_shape=(
            jax.ShapeDtypeStruct((S, B2), jnp.float32),
            jax.ShapeDtypeStruct((S, B2), jnp.int32),
        ),
    )(src, dst, bd)


def _sc_body(fidx_hbm, w_hbm, a_hbm, idx_v, w_v, zbuf,
             a_sh0, a_sh1, sin0, sin1, sz0, sz1, swo0, swo1, ssc):
    c = lax.axis_index("c")
    sid = lax.axis_index("s")
    abufs = (a_sh0, a_sh1)
    sins = (sin0, sin1)
    szs = (sz0, sz1)
    swos = (swo0, swo1)
    myslice = pl.ds(sid * SLICE, SLICE)

    def _in_copies(s, slot, sem):
        return (
            pltpu.make_async_copy(
                fidx_hbm.at[s, pl.ds(sid * CHUNKS, CHUNKS)], idx_v.at[slot], sem
            ),
            pltpu.make_async_copy(
                w_hbm.at[s, pl.ds(sid * CHUNKS, CHUNKS)], w_v.at[slot], sem
            ),
        )

    def _zinit(i, carry):
        zbuf[pl.ds(i * 16, 16)] = jnp.zeros((16,), jnp.float32)
        return carry

    lax.fori_loop(0, SLICE // 16, _zinit, 0)
    pltpu.async_copy(zbuf, a_sh0.at[myslice], sz0)
    pltpu.async_copy(zbuf, a_sh1.at[myslice], sz1)
    for d in _in_copies(c * SPS, 0, sin0):
        d.start()

    def _step(si, b, not_first, not_last):
        ab = abufs[b]
        o = 1 - b
        for d in _in_copies(si, b, sins[b]):
            d.wait()

        @pl.when(not_last)
        def _():
            for d in _in_copies(si + 1, o, sins[o]):
                d.start()

        pltpu.make_async_copy(zbuf, ab.at[myslice], szs[b]).wait()
        plsc.subcore_barrier()
        for j in range(CHUNKS):
            pltpu.async_copy(w_v.at[b, j], ab.at[idx_v.at[b, j]], ssc, add=True)
        for j in range(CHUNKS):
            pltpu.make_async_copy(w_v.at[b, j], ab.at[idx_v.at[b, j]], ssc).wait()
        plsc.subcore_barrier()
        pltpu.async_copy(
            ab.at[myslice],
            a_hbm.at[pl.ds(si * AFLAT + sid * SLICE, SLICE)],
            swos[b],
        )

        @pl.when(not_first)
        def _():
            pltpu.make_async_copy(
                abufs[o].at[myslice],
                a_hbm.at[pl.ds((si - 1) * AFLAT + sid * SLICE, SLICE)],
                swos[o],
            ).wait()

        @pl.when(jnp.logical_and(not_first, not_last))
        def _():
            pltpu.async_copy(zbuf, abufs[o].at[myslice], szs[o])

    def _pair(k, carry):
        si0 = c * SPS + 2 * k
        _step(si0, 0, k > 0, jnp.bool_(True))
        _step(si0 + 1, 1, jnp.bool_(True), k < SPS // 2 - 1)
        return carry

    lax.fori_loop(0, SPS // 2, _pair, 0)
    bl = (SPS - 1) % 2
    pltpu.make_async_copy(
        abufs[bl].at[myslice],
        a_hbm.at[pl.ds((c * SPS + SPS - 1) * AFLAT + sid * SLICE, SLICE)],
        swos[bl],
    ).wait()


def _sc_scatter(fidx, w):
    mesh = plsc.VectorSubcoreMesh(core_axis_name="c", subcore_axis_name="s")
    kfn = functools.partial(
        pl.kernel,
        mesh=mesh,
        out_type=jax.ShapeDtypeStruct((S * AFLAT,), jnp.float32),
        scratch_types=[
            pltpu.VMEM((2, CHUNKS, 128), jnp.int32),
            pltpu.VMEM((2, CHUNKS, 128), jnp.float32),
            pltpu.VMEM((SLICE,), jnp.float32),
            pltpu.VMEM_SHARED((AFLAT,), jnp.float32),
            pltpu.VMEM_SHARED((AFLAT,), jnp.float32),
            pltpu.SemaphoreType.DMA,
            pltpu.SemaphoreType.DMA,
            pltpu.SemaphoreType.DMA,
            pltpu.SemaphoreType.DMA,
            pltpu.SemaphoreType.DMA,
            pltpu.SemaphoreType.DMA,
            pltpu.SemaphoreType.DMA,
        ],
    )(_sc_body)
    return kfn(fidx, w)


SPB = 4


def _gcn_body(x_ref, a_ref, w_ref, b_ref, o_ref):
    for u in range(SPB):
        x = x_ref[u]
        a = a_ref[u * AVROWS:(u + 1) * AVROWS]
        h = jnp.dot(x, w_ref[...], preferred_element_type=jnp.float32)
        hp = jnp.concatenate(
            [h, jnp.zeros((AROWS - N, DIM), jnp.float32)], axis=0
        )
        planes = [a[j * AROWS:(j + 1) * AROWS] for j in range(NPLANES)]
        deg = 1.0 + jnp.sum(sum(planes[1:], planes[0]), axis=1)
        dis = lax.rsqrt(deg)
        t = hp * dis[:, None]
        tp = jnp.concatenate(
            [t, jnp.zeros((ACOLS - AROWS, DIM), jnp.float32)], axis=0
        )
        z = jnp.zeros((AROWS, DIM), jnp.float32)
        for j in range(NPLANES):
            tj = lax.slice(tp, (j * 128, 0), (j * 128 + 128, DIM))
            z = z + jnp.dot(planes[j], tj, preferred_element_type=jnp.float32)
        out = dis[:, None] * (z + t) + b_ref[...]
        o_ref[pl.ds(u * N, N), :] = out[:N]


def _gcn_tc(x_data, a, W, bias2d):
    return pl.pallas_call(
        _gcn_body,
        grid=(S // SPB,),
        in_specs=[
            pl.BlockSpec((SPB, N, DIM), lambda i: (i, 0, 0)),
            pl.BlockSpec((SPB * AVROWS, 128), lambda i: (i, 0)),
            pl.BlockSpec((DIM, DIM), lambda i: (0, 0)),
            pl.BlockSpec((1, DIM), lambda i: (0, 0)),
        ],
        out_specs=pl.BlockSpec((SPB * N, DIM), lambda i: (i, 0)),
        out_shape=jax.ShapeDtypeStruct((S * N, DIM), jnp.float32),
    )(x_data, a, W, bias2d)


@jax.jit
def kernel(x_data, edge_index, bond_dist, W, bias):
    src = edge_index[:, 0, :].astype(jnp.int32)
    dst = edge_index[:, 1, :].astype(jnp.int32)
    w, fidx = _prep(src, dst, bond_dist)
    a_flat = _sc_scatter(
        fidx.reshape(S, B2 // 128, 128), w.reshape(S, B2 // 128, 128)
    )
    a = a_flat.reshape(S * AVROWS, 128)
    return _gcn_tc(x_data, a, W, bias.reshape(1, DIM))


# speedup vs baseline: 2.4399x; 1.0044x over previous
import functools

import jax
import jax.numpy as jnp
from jax import lax
from jax.experimental import pallas as pl
from jax.experimental.pallas import tpu as pltpu
from jax.experimental.pallas import tpu_sc as plsc

S = 128
N = 558
B2 = 4096
DIM = 128
AROWS = 560
ACOLS = 640
NPLANES = ACOLS // 128
AFLAT = AROWS * ACOLS
AVROWS = AFLAT // 128
NTILES = 16
SPS = S // 2
CHUNKS = B2 // NTILES // 128
SLICE = AFLAT // NTILES


def _prep_body(src_ref, dst_ref, bd_ref, w_ref, fidx_ref):
    src = src_ref[...]
    dst = dst_ref[...]
    bd = bd_ref[...]
    neq = (src != dst).astype(jnp.int32)
    b = jnp.sum(neq, axis=1, keepdims=True)
    pos = lax.broadcasted_iota(jnp.int32, (S, B2), 1)
    mask = pos < b
    masked = jnp.where(mask, bd, -jnp.inf)
    m = jnp.max(masked)
    w_ref[...] = jnp.where(mask, bd / m, jnp.zeros_like(bd))
    fidx_ref[...] = (
        (src // 128) * (AROWS * 128) + dst * 128 + (src % 128)
    )


def _prep(src, dst, bd):
    return pl.pallas_call(
        _prep_body,
        out_shape=(
            jax.ShapeDtypeStruct((S, B2), jnp.float32),
            jax.ShapeDtypeStruct((S, B2), jnp.int32),
        ),
    )(src, dst, bd)


def _sc_body(fidx_hbm, w_hbm, a_hbm, idx_v, w_v, negw_v, zbuf,
             a_sh0, a_sh1, sin0, sin1, sin2, sin3, swo0, swo1, ssc, sneg):
    c = lax.axis_index("c")
    sid = lax.axis_index("s")
    abufs = (a_sh0, a_sh1)
    sins = (sin0, sin1, sin2, sin3)
    swos = (swo0, swo1)
    myslice = pl.ds(sid * SLICE, SLICE)

    def _in_copies(s, slot, sem):
        return (
            pltpu.make_async_copy(
                fidx_hbm.at[s, pl.ds(sid * CHUNKS, CHUNKS)], idx_v.at[slot], sem
            ),
            pltpu.make_async_copy(
                w_hbm.at[s, pl.ds(sid * CHUNKS, CHUNKS)], w_v.at[slot], sem
            ),
        )

    def _scat(values, slot, b, sem):
        return [
            pltpu.make_async_copy(
                values.at[slot, j], abufs[b].at[idx_v.at[slot, j]], sem
            )
            for j in range(CHUNKS)
        ]

    def _zinit(i, carry):
        zbuf[pl.ds(i * 16, 16)] = jnp.zeros((16,), jnp.float32)
        return carry

    lax.fori_loop(0, SLICE // 16, _zinit, 0)
    pltpu.sync_copy(zbuf, a_sh0.at[myslice])
    pltpu.sync_copy(zbuf, a_sh1.at[myslice])
    for d in _in_copies(c * SPS, 0, sin0):
        d.start()

    def _step(si, slot, not_first, not_last, neg_fired):
        b = slot % 2
        o = 1 - b
        pslot = (slot + 3) % 4
        ppslot = (slot + 2) % 4
        ab = abufs[b]
        for d in _in_copies(si, slot, sins[slot]):
            d.wait()

        @pl.when(not_last)
        def _():
            for d in _in_copies(si + 1, (slot + 1) % 4, sins[(slot + 1) % 4]):
                d.start()

        for j in range(CHUNKS):
            for k in range(128 // 16):
                negw_v[slot, j, pl.ds(k * 16, 16)] = (
                    -w_v[slot, j, pl.ds(k * 16, 16)]
                )
        for j in range(CHUNKS):
            pltpu.async_copy(
                w_v.at[slot, j], ab.at[idx_v.at[slot, j]], ssc, add=True
            )
        for d in _scat(w_v, slot, b, ssc):
            d.wait()

        @pl.when(neg_fired)
        def _():
            for d in _scat(negw_v, ppslot, b, sneg):
                d.wait()

        plsc.subcore_barrier()
        pltpu.async_copy(
            ab.at[myslice],
            a_hbm.at[pl.ds(si * AFLAT + sid * SLICE, SLICE)],
            swos[b],
        )

        @pl.when(not_first)
        def _():
            pltpu.make_async_copy(
                abufs[o].at[myslice],
                a_hbm.at[pl.ds((si - 1) * AFLAT + sid * SLICE, SLICE)],
                swos[o],
            ).wait()
            for j in range(CHUNKS):
                pltpu.async_copy(
                    negw_v.at[pslot, j], abufs[o].at[idx_v.at[pslot, j]],
                    sneg, add=True,
                )

    def _quad(k, carry):
        si0 = c * SPS + 4 * k
        _step(si0, 0, k > 0, jnp.bool_(True), k > 0)
        _step(si0 + 1, 1, jnp.bool_(True), jnp.bool_(True), k > 0)
        _step(si0 + 2, 2, jnp.bool_(True), jnp.bool_(True), jnp.bool_(True))
        _step(si0 + 3, 3, jnp.bool_(True), k < SPS // 4 - 1, jnp.bool_(True))
        return carry

    lax.fori_loop(0, SPS // 4, _quad, 0)
    bl = (SPS - 1) % 2
    pltpu.make_async_copy(
        abufs[bl].at[myslice],
        a_hbm.at[pl.ds((c * SPS + SPS - 1) * AFLAT + sid * SLICE, SLICE)],
        swos[bl],
    ).wait()
    lastp = (SPS - 2) % 4
    for d in _scat(negw_v, lastp, (SPS - 2) % 2, sneg):
        d.wait()


def _sc_scatter(fidx, w):
    mesh = plsc.VectorSubcoreMesh(core_axis_name="c", subcore_axis_name="s")
    kfn = functools.partial(
        pl.kernel,
        mesh=mesh,
        out_type=jax.ShapeDtypeStruct((S * AFLAT,), jnp.float32),
        scratch_types=[
            pltpu.VMEM((4, CHUNKS, 128), jnp.int32),
            pltpu.VMEM((4, CHUNKS, 128), jnp.float32),
            pltpu.VMEM((4, CHUNKS, 128), jnp.float32),
            pltpu.VMEM((SLICE,), jnp.float32),
            pltpu.VMEM_SHARED((AFLAT,), jnp.float32),
            pltpu.VMEM_SHARED((AFLAT,), jnp.float32),
            pltpu.SemaphoreType.DMA,
            pltpu.SemaphoreType.DMA,
            pltpu.SemaphoreType.DMA,
            pltpu.SemaphoreType.DMA,
            pltpu.SemaphoreType.DMA,
            pltpu.SemaphoreType.DMA,
            pltpu.SemaphoreType.DMA,
            pltpu.SemaphoreType.DMA,
        ],
    )(_sc_body)
    return kfn(fidx, w)


SPB = 4


def _gcn_body(x_ref, a_ref, w_ref, b_ref, o_ref):
    for u in range(SPB):
        x = x_ref[u]
        a = a_ref[u * AVROWS:(u + 1) * AVROWS]
        h = jnp.dot(x, w_ref[...], preferred_element_type=jnp.float32)
        hp = jnp.concatenate(
            [h, jnp.zeros((AROWS - N, DIM), jnp.float32)], axis=0
        )
        planes = [a[j * AROWS:(j + 1) * AROWS] for j in range(NPLANES)]
        deg = 1.0 + jnp.sum(sum(planes[1:], planes[0]), axis=1)
        dis = lax.rsqrt(deg)
        t = hp * dis[:, None]
        tp = jnp.concatenate(
            [t, jnp.zeros((ACOLS - AROWS, DIM), jnp.float32)], axis=0
        )
        z = jnp.zeros((AROWS, DIM), jnp.float32)
        for j in range(NPLANES):
            tj = lax.slice(tp, (j * 128, 0), (j * 128 + 128, DIM))
            z = z + jnp.dot(planes[j], tj, preferred_element_type=jnp.float32)
        out = dis[:, None] * (z + t) + b_ref[...]
        o_ref[pl.ds(u * N, N), :] = out[:N]


def _gcn_tc(x_data, a, W, bias2d):
    return pl.pallas_call(
        _gcn_body,
        grid=(S // SPB,),
        in_specs=[
            pl.BlockSpec((SPB, N, DIM), lambda i: (i, 0, 0)),
            pl.BlockSpec((SPB * AVROWS, 128), lambda i: (i, 0)),
            pl.BlockSpec((DIM, DIM), lambda i: (0, 0)),
            pl.BlockSpec((1, DIM), lambda i: (0, 0)),
        ],
        out_specs=pl.BlockSpec((SPB * N, DIM), lambda i: (i, 0)),
        out_shape=jax.ShapeDtypeStruct((S * N, DIM), jnp.float32),
    )(x_data, a, W, bias2d)


@jax.jit
def kernel(x_data, edge_index, bond_dist, W, bias):
    src = edge_index[:, 0, :].astype(jnp.int32)
    dst = edge_index[:, 1, :].astype(jnp.int32)
    w, fidx = _prep(src, dst, bond_dist)
    a_flat = _sc_scatter(
        fidx.reshape(S, B2 // 128, 128), w.reshape(S, B2 // 128, 128)
    )
    a = a_flat.reshape(S * AVROWS, 128)
    return _gcn_tc(x_data, a, W, bias.reshape(1, DIM))


# speedup vs baseline: 2.4535x; 1.0056x over previous
import functools

import jax
import jax.numpy as jnp
from jax import lax
from jax.experimental import pallas as pl
from jax.experimental.pallas import tpu as pltpu
from jax.experimental.pallas import tpu_sc as plsc

S = 128
N = 558
B2 = 4096
DIM = 128
AROWS = 560
ACOLS = 640
NPLANES = ACOLS // 128
AFLAT = AROWS * ACOLS
AVROWS = AFLAT // 128
NTILES = 16
SPS = S // 2
CHUNKS = B2 // NTILES // 128
SLICE = AFLAT // NTILES


def _prep_body(src_ref, dst_ref, bd_ref, w_ref, fidx_ref):
    src = src_ref[...]
    dst = dst_ref[...]
    bd = bd_ref[...]
    neq = (src != dst).astype(jnp.int32)
    b = jnp.sum(neq, axis=1, keepdims=True)
    pos = lax.broadcasted_iota(jnp.int32, (S, B2), 1)
    mask = pos < b
    masked = jnp.where(mask, bd, -jnp.inf)
    m = jnp.max(masked)
    w_ref[...] = jnp.where(mask, bd / m, jnp.zeros_like(bd))
    fidx_ref[...] = (
        (src // 128) * (AROWS * 128) + dst * 128 + (src % 128)
    )


def _prep(src, dst, bd):
    return pl.pallas_call(
        _prep_body,
        out_shape=(
            jax.ShapeDtypeStruct((S, B2), jnp.float32),
            jax.ShapeDtypeStruct((S, B2), jnp.int32),
        ),
    )(src, dst, bd)


def _sc_body(fidx_hbm, w_hbm, a_hbm, idx_v, w_v, negw_v, zbuf,
             a_sh0, a_sh1, sin0, sin1, sin2, sin3, swo0, swo1, ssc, sneg):
    c = lax.axis_index("c")
    sid = lax.axis_index("s")
    abufs = (a_sh0, a_sh1)
    sins = (sin0, sin1, sin2, sin3)
    swos = (swo0, swo1)
    myslice = pl.ds(sid * SLICE, SLICE)

    def _in_copies(s, slot, sem):
        return (
            pltpu.make_async_copy(
                fidx_hbm.at[s, pl.ds(sid * CHUNKS, CHUNKS)], idx_v.at[slot], sem
            ),
            pltpu.make_async_copy(
                w_hbm.at[s, pl.ds(sid * CHUNKS, CHUNKS)], w_v.at[slot], sem
            ),
        )

    def _scat(values, slot, b, sem):
        return [
            pltpu.make_async_copy(
                values.at[slot, j], abufs[b].at[idx_v.at[slot, j]], sem
            )
            for j in range(CHUNKS)
        ]

    def _zinit(i, carry):
        zbuf[pl.ds(i * 16, 16)] = jnp.zeros((16,), jnp.float32)
        return carry

    lax.fori_loop(0, SLICE // 16, _zinit, 0)
    pltpu.sync_copy(zbuf, a_sh0.at[myslice])
    pltpu.sync_copy(zbuf, a_sh1.at[myslice])
    for d in _in_copies(c * SPS, 0, sin0):
        d.start()

    def _step(si, slot, not_last, have_prev2):
        b = slot % 2
        ppslot = (slot + 2) % 4
        ab = abufs[b]
        for d in _in_copies(si, slot, sins[slot]):
            d.wait()

        @pl.when(not_last)
        def _():
            for d in _in_copies(si + 1, (slot + 1) % 4, sins[(slot + 1) % 4]):
                d.start()

        for j in range(CHUNKS):
            for k in range(128 // 16):
                negw_v[slot, j, pl.ds(k * 16, 16)] = (
                    -w_v[slot, j, pl.ds(k * 16, 16)]
                )

        @pl.when(have_prev2)
        def _():
            pltpu.make_async_copy(
                ab.at[myslice],
                a_hbm.at[pl.ds((si - 2) * AFLAT + sid * SLICE, SLICE)],
                swos[b],
            ).wait()

        plsc.subcore_barrier()

        @pl.when(have_prev2)
        def _():
            for j in range(CHUNKS):
                pltpu.async_copy(
                    negw_v.at[ppslot, j], ab.at[idx_v.at[ppslot, j]],
                    sneg, add=True,
                )
        for j in range(CHUNKS):
            pltpu.async_copy(
                w_v.at[slot, j], ab.at[idx_v.at[slot, j]], ssc, add=True
            )
        for d in _scat(w_v, slot, b, ssc):
            d.wait()

        @pl.when(have_prev2)
        def _():
            for d in _scat(negw_v, ppslot, b, sneg):
                d.wait()

        plsc.subcore_barrier()
        pltpu.async_copy(
            ab.at[myslice],
            a_hbm.at[pl.ds(si * AFLAT + sid * SLICE, SLICE)],
            swos[b],
        )

    def _quad(k, carry):
        si0 = c * SPS + 4 * k
        _step(si0, 0, jnp.bool_(True), k > 0)
        _step(si0 + 1, 1, jnp.bool_(True), k > 0)
        _step(si0 + 2, 2, jnp.bool_(True), jnp.bool_(True))
        _step(si0 + 3, 3, k < SPS // 4 - 1, jnp.bool_(True))
        return carry

    lax.fori_loop(0, SPS // 4, _quad, 0)
    for back in (2, 1):
        sl = c * SPS + SPS - back
        bb = (SPS - back) % 2
        pltpu.make_async_copy(
            abufs[bb].at[myslice],
            a_hbm.at[pl.ds(sl * AFLAT + sid * SLICE, SLICE)],
            swos[bb],
        ).wait()


def _sc_scatter(fidx, w):
    mesh = plsc.VectorSubcoreMesh(core_axis_name="c", subcore_axis_name="s")
    kfn = functools.partial(
        pl.kernel,
        mesh=mesh,
        out_type=jax.ShapeDtypeStruct((S * AFLAT,), jnp.float32),
        scratch_types=[
            pltpu.VMEM((4, CHUNKS, 128), jnp.int32),
            pltpu.VMEM((4, CHUNKS, 128), jnp.float32),
            pltpu.VMEM((4, CHUNKS, 128), jnp.float32),
            pltpu.VMEM((SLICE,), jnp.float32),
            pltpu.VMEM_SHARED((AFLAT,), jnp.float32),
            pltpu.VMEM_SHARED((AFLAT,), jnp.float32),
            pltpu.SemaphoreType.DMA,
            pltpu.SemaphoreType.DMA,
            pltpu.SemaphoreType.DMA,
            pltpu.SemaphoreType.DMA,
            pltpu.SemaphoreType.DMA,
            pltpu.SemaphoreType.DMA,
            pltpu.SemaphoreType.DMA,
            pltpu.SemaphoreType.DMA,
        ],
    )(_sc_body)
    return kfn(fidx, w)


SPB = 4


def _gcn_body(x_ref, a_ref, w_ref, b_ref, o_ref):
    for u in range(SPB):
        x = x_ref[u]
        a = a_ref[u * AVROWS:(u + 1) * AVROWS]
        h = jnp.dot(x, w_ref[...], preferred_element_type=jnp.float32)
        hp = jnp.concatenate(
            [h, jnp.zeros((AROWS - N, DIM), jnp.float32)], axis=0
        )
        planes = [a[j * AROWS:(j + 1) * AROWS] for j in range(NPLANES)]
        deg = 1.0 + jnp.sum(sum(planes[1:], planes[0]), axis=1)
        dis = lax.rsqrt(deg)
        t = hp * dis[:, None]
        tp = jnp.concatenate(
            [t, jnp.zeros((ACOLS - AROWS, DIM), jnp.float32)], axis=0
        )
        z = jnp.zeros((AROWS, DIM), jnp.float32)
        for j in range(NPLANES):
            tj = lax.slice(tp, (j * 128, 0), (j * 128 + 128, DIM))
            z = z + jnp.dot(planes[j], tj, preferred_element_type=jnp.float32)
        out = dis[:, None] * (z + t) + b_ref[...]
        o_ref[pl.ds(u * N, N), :] = out[:N]


def _gcn_tc(x_data, a, W, bias2d):
    return pl.pallas_call(
        _gcn_body,
        grid=(S // SPB,),
        in_specs=[
            pl.BlockSpec((SPB, N, DIM), lambda i: (i, 0, 0)),
            pl.BlockSpec((SPB * AVROWS, 128), lambda i: (i, 0)),
            pl.BlockSpec((DIM, DIM), lambda i: (0, 0)),
            pl.BlockSpec((1, DIM), lambda i: (0, 0)),
        ],
        out_specs=pl.BlockSpec((SPB * N, DIM), lambda i: (i, 0)),
        out_shape=jax.ShapeDtypeStruct((S * N, DIM), jnp.float32),
    )(x_data, a, W, bias2d)


@jax.jit
def kernel(x_data, edge_index, bond_dist, W, bias):
    src = edge_index[:, 0, :].astype(jnp.int32)
    dst = edge_index[:, 1, :].astype(jnp.int32)
    w, fidx = _prep(src, dst, bond_dist)
    a_flat = _sc_scatter(
        fidx.reshape(S, B2 // 128, 128), w.reshape(S, B2 // 128, 128)
    )
    a = a_flat.reshape(S * AVROWS, 128)
    return _gcn_tc(x_data, a, W, bias.reshape(1, DIM))
